# baseline (speedup 1.0000x reference)
import functools

import jax
import jax.numpy as jnp
from jax import lax
from jax.experimental import pallas as pl
from jax.experimental.pallas import tpu as pltpu
from jax.experimental.pallas import tpu_sc as plsc

NC = 2
NS = 16
NW = NC * NS
CHUNK = 128
NBUF = 3


def _sc_mesh():
  return plsc.VectorSubcoreMesh(
      core_axis_name="c", subcore_axis_name="s",
      num_cores=NC, num_subcores=NS)


def _zero16():
  return jnp.zeros((16,), jnp.float32)


def _zero_table(sid, zero_v, acc_sh, n, zrows):
  zchunks = n // zrows
  ziters = -(-zchunks // NS)

  def zbody(i, carry):
    c = sid + NS * i

    @pl.when(c < zchunks)
    def _():
      pltpu.sync_copy(zero_v, acc_sh.at[pl.ds(c * zrows, zrows)])
    return carry

  lax.fori_loop(0, ziters, zbody, 0)


def _copy_out(sid, cid, acc_sh, out_hbm, n, crows):
  cchunks = n // crows
  citers = -(-cchunks // NS)

  def obody(i, carry):
    c = sid + NS * i

    @pl.when(c < cchunks)
    def _():
      pltpu.sync_copy(acc_sh.at[pl.ds(c * crows, crows)],
                      out_hbm.at[pl.ds(cid * n + c * crows, crows)])
    return carry

  lax.fori_loop(0, citers, obody, 0)


def _deg_sc(dstp, ones2d, n):
  d = ones2d.shape[1]
  ep = dstp.shape[0]
  e_half = ep // NC
  cpt = e_half // (NS * CHUNK)
  zrows = 8

  @functools.partial(
      pl.kernel,
      out_type=jax.ShapeDtypeStruct((NC * n, d), jnp.float32),
      mesh=_sc_mesh(),
      scratch_types=(
          [pltpu.VMEM((CHUNK,), jnp.int32) for _ in range(NBUF)] +
          [pltpu.VMEM((CHUNK, d), jnp.float32),
           pltpu.VMEM((zrows, d), jnp.float32),
           pltpu.VMEM_SHARED((n + 8, d), jnp.float32)] +
          [pltpu.SemaphoreType.DMA for _ in range(NBUF)]),
  )
  def k(dst_hbm, ones_hbm, out_hbm, *s):
    dst_v = s[:NBUF]
    ones_v, zero_v, acc_sh = s[NBUF:NBUF + 3]
    s_di = s[NBUF + 3:]
    cid = lax.axis_index("c")
    sid = lax.axis_index("s")
    z16 = _zero16()
    for r in range(zrows):
      for j in range(d // 16):
        zero_v[r, pl.ds(16 * j, 16)] = z16
    pltpu.sync_copy(ones_hbm, ones_v)
    _zero_table(sid, zero_v, acc_sh, n, zrows)
    plsc.subcore_barrier()

    base = cid * e_half + sid * cpt * CHUNK
    di = {}

    def issue_idx(i):
      b = i % NBUF
      di[i] = pltpu.async_copy(
          dst_hbm.at[pl.ds(base + i * CHUNK, CHUNK)], dst_v[b], s_di[b])

    issue_idx(0)
    if cpt > 1:
      issue_idx(1)
    for i in range(cpt):
      di[i].wait()
      pltpu.sync_copy(ones_v, acc_sh.at[dst_v[i % NBUF]], add=True)
      if i + 2 < cpt:
        issue_idx(i + 2)

    plsc.subcore_barrier()
    _copy_out(sid, cid, acc_sh, out_hbm, n, 40)

  return k(dstp, ones2d)


def _agg_sc(y, srcp, dstp):
  n, d = y.shape
  ep = srcp.shape[0]
  e_half = ep // NC
  cpt = e_half // (NS * CHUNK)
  zrows = 8

  @functools.partial(
      pl.kernel,
      out_type=jax.ShapeDtypeStruct((NC * n, d), jnp.float32),
      mesh=_sc_mesh(),
      scratch_types=(
          [pltpu.VMEM((CHUNK,), jnp.int32) for _ in range(2 * NBUF)] +
          [pltpu.VMEM((CHUNK, d), jnp.float32) for _ in range(NBUF)] +
          [pltpu.VMEM((zrows, d), jnp.float32),
           pltpu.VMEM_SHARED((n + 8, d), jnp.float32)] +
          [pltpu.SemaphoreType.DMA for _ in range(3 * NBUF)]),
  )
  def k(y_hbm, src_hbm, dst_hbm, out_hbm, *s):
    src_v = s[:NBUF]
    dst_v = s[NBUF:2 * NBUF]
    rows_v = s[2 * NBUF:3 * NBUF]
    zero_v, acc_sh = s[3 * NBUF:3 * NBUF + 2]
    s_si = s[3 * NBUF + 2:4 * NBUF + 2]
    s_di = s[4 * NBUF + 2:5 * NBUF + 2]
    s_g = s[5 * NBUF + 2:]
    cid = lax.axis_index("c")
    sid = lax.axis_index("s")
    z16 = _zero16()
    for r in range(zrows):
      for j in range(d // 16):
        zero_v[r, pl.ds(16 * j, 16)] = z16
    _zero_table(sid, zero_v, acc_sh, n, zrows)
    plsc.subcore_barrier()

    base = cid * e_half + sid * cpt * CHUNK
    di = {}
    gi = {}

    def issue_idx(i):
      b = i % NBUF
      off = base + i * CHUNK
      di[i] = (
          pltpu.async_copy(src_hbm.at[pl.ds(off, CHUNK)], src_v[b], s_si[b]),
          pltpu.async_copy(dst_hbm.at[pl.ds(off, CHUNK)], dst_v[b], s_di[b]))

    def issue_gather(i):
      b = i % NBUF
      gi[i] = pltpu.async_copy(y_hbm.at[src_v[b]], rows_v[b], s_g[b])

    issue_idx(0)
    if cpt > 1:
      issue_idx(1)
    di[0][0].wait()
    issue_gather(0)
    for i in range(cpt):
      if i + 1 < cpt:
        di[i + 1][0].wait()
        issue_gather(i + 1)
      gi[i].wait()
      di[i][1].wait()
      pltpu.sync_copy(rows_v[i % NBUF], acc_sh.at[dst_v[i % NBUF]], add=True)
      if i + 2 < cpt:
        issue_idx(i + 2)

    plsc.subcore_barrier()
    _copy_out(sid, cid, acc_sh, out_hbm, n, 40)

  return k(y, srcp, dstp)


def _decode_sc(z, ai, bi):
  n, d = z.shape
  elp = ai.shape[0]
  per_tile = elp // NW
  kc = 112
  cpt = per_tile // kc
  nj = d // 16
  nb = 2

  @functools.partial(
      pl.kernel,
      out_type=jax.ShapeDtypeStruct((elp, 16), jnp.float32),
      mesh=_sc_mesh(),
      scratch_types=(
          [pltpu.VMEM((kc,), jnp.int32) for _ in range(2 * nb)] +
          [pltpu.VMEM((kc, d), jnp.float32) for _ in range(2 * nb)] +
          [pltpu.VMEM((kc, 16), jnp.float32) for _ in range(nb)] +
          [pltpu.SemaphoreType.DMA for _ in range(5 * nb)]),
  )
  def k(z_hbm, a_hbm, b_hbm, out_hbm, *s):
    ai_v = s[:nb]
    bi_v = s[nb:2 * nb]
    za_v = s[2 * nb:3 * nb]
    zb_v = s[3 * nb:4 * nb]
    part_v = s[4 * nb:5 * nb]
    s_ai = s[5 * nb:6 * nb]
    s_bi = s[6 * nb:7 * nb]
    s_ga = s[7 * nb:8 * nb]
    s_gb = s[8 * nb:9 * nb]
    s_out = s[9 * nb:]
    cid = lax.axis_index("c")
    sid = lax.axis_index("s")
    wid = sid * NC + cid
    base = wid * per_tile
    di = {}
    gi = {}
    oi = {}

    def issue_idx(i):
      b = i % nb
      off = base + i * kc
      di[i] = (
          pltpu.async_copy(a_hbm.at[pl.ds(off, kc)], ai_v[b], s_ai[b]),
          pltpu.async_copy(b_hbm.at[pl.ds(off, kc)], bi_v[b], s_bi[b]))

    def issue_gathers(i):
      b = i % nb
      gi[i] = (
          pltpu.async_copy(z_hbm.at[ai_v[b]], za_v[b], s_ga[b]),
          pltpu.async_copy(z_hbm.at[bi_v[b]], zb_v[b], s_gb[b]))

    issue_idx(0)
    if cpt > 1:
      issue_idx(1)
    di[0][0].wait()
    di[0][1].wait()
    issue_gathers(0)
    for i in range(cpt):
      b = i % nb
      if i + 1 < cpt:
        di[i + 1][0].wait()
        di[i + 1][1].wait()
        issue_gathers(i + 1)
      gi[i][0].wait()
      gi[i][1].wait()
      if i - nb >= 0:
        oi[i - nb].wait()

      def ebody(ei, ecarry):
        acc = za_v[b][ei, pl.ds(0, 16)] * zb_v[b][ei, pl.ds(0, 16)]
        for j in range(1, nj):
          acc = acc + (za_v[b][ei, pl.ds(16 * j, 16)] *
                       zb_v[b][ei, pl.ds(16 * j, 16)])
        part_v[b][ei, :] = acc
        return ecarry

      lax.fori_loop(0, kc, ebody, 0)
      oi[i] = pltpu.async_copy(
          part_v[b], out_hbm.at[pl.ds(base + i * kc, kc)], s_out[b])
      if i + 2 < cpt:
        issue_idx(i + 2)
    for i in range(max(cpt - nb, 0), cpt):
      oi[i].wait()

  return k(z, ai, bi)


def _tc_mm1(deg2, x, w1):
  n, d = x.shape
  rb = 1000
  g = n // rb

  def body(dega, degb, x_ref, w_ref, y_ref):
    deg = dega[:, 0:1] + degb[:, 0:1] + 1.0
    dinv = lax.rsqrt(deg)
    y_ref[...] = dinv * jnp.dot(x_ref[...], w_ref[...],
                                preferred_element_type=jnp.float32)

  return pl.pallas_call(
      body,
      grid=(g,),
      in_specs=[
          pl.BlockSpec((rb, d), lambda i: (i, 0)),
          pl.BlockSpec((rb, d), lambda i: (i + g, 0)),
          pl.BlockSpec((rb, d), lambda i: (i, 0)),
          pl.BlockSpec((d, d), lambda i: (0, 0)),
      ],
      out_specs=pl.BlockSpec((rb, d), lambda i: (i, 0)),
      out_shape=jax.ShapeDtypeStruct((n, d), jnp.float32),
  )(deg2, deg2, x, w1)


def _tc_mm2(deg2, acc2, y1, b1, w2):
  n, d = y1.shape
  rb = 1000
  g = n // rb

  def body(dega, degb, acca, accb, y_ref, b_ref, w_ref, out_ref):
    deg = dega[:, 0:1] + degb[:, 0:1] + 1.0
    dinv = lax.rsqrt(deg)
    sums = (acca[...] + accb[...] + y_ref[...]) * dinv + b_ref[...]
    h = jnp.maximum(sums, 0.0)
    out_ref[...] = dinv * jnp.dot(h, w_ref[...],
                                  preferred_element_type=jnp.float32)

  return pl.pallas_call(
      body,
      grid=(g,),
      in_specs=[
          pl.BlockSpec((rb, d), lambda i: (i, 0)),
          pl.BlockSpec((rb, d), lambda i: (i + g, 0)),
          pl.BlockSpec((rb, d), lambda i: (i, 0)),
          pl.BlockSpec((rb, d), lambda i: (i + g, 0)),
          pl.BlockSpec((rb, d), lambda i: (i, 0)),
          pl.BlockSpec((1, d), lambda i: (0, 0)),
          pl.BlockSpec((d, d), lambda i: (0, 0)),
      ],
      out_specs=pl.BlockSpec((rb, d), lambda i: (i, 0)),
      out_shape=jax.ShapeDtypeStruct((n, d), jnp.float32),
  )(deg2, deg2, acc2, acc2, y1, b1, w2)


def _tc_fin(deg2, acc2, y2, b2):
  n, d = y2.shape
  rb = 1000
  g = n // rb

  def body(dega, degb, acca, accb, y_ref, b_ref, out_ref):
    deg = dega[:, 0:1] + degb[:, 0:1] + 1.0
    dinv = lax.rsqrt(deg)
    out_ref[...] = (acca[...] + accb[...] + y_ref[...]) * dinv + b_ref[...]

  return pl.pallas_call(
      body,
      grid=(g,),
      in_specs=[
          pl.BlockSpec((rb, d), lambda i: (i, 0)),
          pl.BlockSpec((rb, d), lambda i: (i + g, 0)),
          pl.BlockSpec((rb, d), lambda i: (i, 0)),
          pl.BlockSpec((rb, d), lambda i: (i + g, 0)),
          pl.BlockSpec((rb, d), lambda i: (i, 0)),
          pl.BlockSpec((1, d), lambda i: (0, 0)),
      ],
      out_specs=pl.BlockSpec((rb, d), lambda i: (i, 0)),
      out_shape=jax.ShapeDtypeStruct((n, d), jnp.float32),
  )(deg2, deg2, acc2, acc2, y2, b2)


def _tc_lanesum(part):
  elp = part.shape[0]
  g = 32
  rb = elp // g

  def body(p_ref, out_ref):
    out_ref[...] = jnp.sum(p_ref[...], axis=1, keepdims=True)

  return pl.pallas_call(
      body,
      grid=(g,),
      in_specs=[pl.BlockSpec((rb, 16), lambda i: (i, 0))],
      out_specs=pl.BlockSpec((rb, 1), lambda i: (i, 0)),
      out_shape=jax.ShapeDtypeStruct((elp, 1), jnp.float32),
  )(part)


def kernel(x, edge_index, edge_label_index, W1, b1, W2, b2):
  n, d = x.shape
  e = edge_index.shape[1]
  el = edge_label_index.shape[1]
  src = edge_index[0].astype(jnp.int32)
  dst = edge_index[1].astype(jnp.int32)

  cpt = -(-e // (NW * CHUNK))
  ep = cpt * NW * CHUNK
  epad = ep - e
  srcp = jnp.concatenate([src, jnp.zeros((epad,), jnp.int32)])
  dstp = jnp.concatenate([dst, jnp.full((epad,), n, jnp.int32)])

  kc = 112
  per_tile = -(-el // NW)
  per_tile = -(-per_tile // kc) * kc
  elp = per_tile * NW
  pad = elp - el
  ai = jnp.concatenate(
      [edge_label_index[0].astype(jnp.int32), jnp.zeros((pad,), jnp.int32)])
  bi = jnp.concatenate(
      [edge_label_index[1].astype(jnp.int32), jnp.zeros((pad,), jnp.int32)])

  ones2d = jnp.ones((CHUNK, d), jnp.float32)
  deg2 = _deg_sc(dstp, ones2d, n)
  y1 = _tc_mm1(deg2, x, W1)
  acc1 = _agg_sc(y1, srcp, dstp)
  y2 = _tc_mm2(deg2, acc1, y1, b1.reshape(1, d), W2)
  acc2 = _agg_sc(y2, srcp, dstp)
  z = _tc_fin(deg2, acc2, y2, b2.reshape(1, d))
  part = _decode_sc(z, ai, bi)
  logits = _tc_lanesum(part)
  return logits[:el, 0]


# speedup vs baseline: 1.5655x; 1.5655x over previous
import functools

import jax
import jax.numpy as jnp
from jax import lax
from jax.experimental import pallas as pl
from jax.experimental.pallas import tpu as pltpu
from jax.experimental.pallas import tpu_sc as plsc

NC = 2
NS = 16
NW = NC * NS
CHUNK = 128
NBUF = 3


def _sc_mesh():
  return plsc.VectorSubcoreMesh(
      core_axis_name="c", subcore_axis_name="s",
      num_cores=NC, num_subcores=NS)


def _zero16():
  return jnp.zeros((16,), jnp.float32)


def _zero_table(sid, zero_v, acc_sh, n, zrows):
  zchunks = n // zrows
  ziters = -(-zchunks // NS)

  def zbody(i, carry):
    c = sid + NS * i

    @pl.when(c < zchunks)
    def _():
      pltpu.sync_copy(zero_v, acc_sh.at[pl.ds(c * zrows, zrows)])
    return carry

  lax.fori_loop(0, ziters, zbody, 0)


def _copy_out(sid, cid, acc_sh, out_hbm, n, crows):
  cchunks = n // crows
  citers = -(-cchunks // NS)

  def obody(i, carry):
    c = sid + NS * i

    @pl.when(c < cchunks)
    def _():
      pltpu.sync_copy(acc_sh.at[pl.ds(c * crows, crows)],
                      out_hbm.at[pl.ds(cid * n + c * crows, crows)])
    return carry

  lax.fori_loop(0, citers, obody, 0)


def _deg_sc(dstp, ones2d, n):
  d = ones2d.shape[1]
  ep = dstp.shape[0]
  e_half = ep // NC
  cpt = e_half // (NS * CHUNK)
  zrows = 8

  @functools.partial(
      pl.kernel,
      out_type=jax.ShapeDtypeStruct((NC * n, d), jnp.float32),
      mesh=_sc_mesh(),
      scratch_types=(
          [pltpu.VMEM((CHUNK,), jnp.int32) for _ in range(NBUF)] +
          [pltpu.VMEM((CHUNK, d), jnp.float32),
           pltpu.VMEM((zrows, d), jnp.float32),
           pltpu.VMEM_SHARED((n + 8, d), jnp.float32)] +
          [pltpu.SemaphoreType.DMA for _ in range(NBUF)]),
  )
  def k(dst_hbm, ones_hbm, out_hbm, *s):
    dst_v = s[:NBUF]
    ones_v, zero_v, acc_sh = s[NBUF:NBUF + 3]
    s_di = s[NBUF + 3:]
    cid = lax.axis_index("c")
    sid = lax.axis_index("s")
    z16 = _zero16()
    for r in range(zrows):
      for j in range(d // 16):
        zero_v[r, pl.ds(16 * j, 16)] = z16
    pltpu.sync_copy(ones_hbm, ones_v)
    _zero_table(sid, zero_v, acc_sh, n, zrows)
    plsc.subcore_barrier()

    base = cid * e_half + sid * cpt * CHUNK
    di = {}

    def issue_idx(i):
      b = i % NBUF
      di[i] = pltpu.async_copy(
          dst_hbm.at[pl.ds(base + i * CHUNK, CHUNK)], dst_v[b], s_di[b])

    issue_idx(0)
    if cpt > 1:
      issue_idx(1)
    for i in range(cpt):
      di[i].wait()
      pltpu.sync_copy(ones_v, acc_sh.at[dst_v[i % NBUF]], add=True)
      if i + 2 < cpt:
        issue_idx(i + 2)

    plsc.subcore_barrier()
    _copy_out(sid, cid, acc_sh, out_hbm, n, 40)

  return k(dstp, ones2d)


def _agg_sc(y, srcp, dstp, n):
  d = y.shape[1]
  ep = srcp.shape[0]
  e_half = ep // NC
  cpt = e_half // (NS * CHUNK)
  zrows = 8

  @functools.partial(
      pl.kernel,
      out_type=jax.ShapeDtypeStruct((NC * n, d), jnp.float32),
      mesh=_sc_mesh(),
      scratch_types=(
          [pltpu.VMEM((CHUNK,), jnp.int32) for _ in range(2 * NBUF)] +
          [pltpu.VMEM((CHUNK, d), jnp.float32) for _ in range(NBUF)] +
          [pltpu.VMEM((zrows, d), jnp.float32),
           pltpu.VMEM_SHARED((n, d), jnp.float32)] +
          [pltpu.SemaphoreType.DMA for _ in range(3 * NBUF)]),
  )
  def k(y_hbm, src_hbm, dst_hbm, out_hbm, *s):
    src_v = s[:NBUF]
    dst_v = s[NBUF:2 * NBUF]
    rows_v = s[2 * NBUF:3 * NBUF]
    zero_v, acc_sh = s[3 * NBUF:3 * NBUF + 2]
    s_si = s[3 * NBUF + 2:4 * NBUF + 2]
    s_di = s[4 * NBUF + 2:5 * NBUF + 2]
    s_g = s[5 * NBUF + 2:]
    cid = lax.axis_index("c")
    sid = lax.axis_index("s")
    z16 = _zero16()
    for r in range(zrows):
      for j in range(d // 16):
        zero_v[r, pl.ds(16 * j, 16)] = z16
    _zero_table(sid, zero_v, acc_sh, n, zrows)
    plsc.subcore_barrier()

    base = cid * e_half + sid * cpt * CHUNK
    di = {}
    gi = {}

    def issue_idx(i):
      b = i % NBUF
      off = base + i * CHUNK
      di[i] = (
          pltpu.async_copy(src_hbm.at[pl.ds(off, CHUNK)], src_v[b], s_si[b]),
          pltpu.async_copy(dst_hbm.at[pl.ds(off, CHUNK)], dst_v[b], s_di[b]))

    def issue_gather(i):
      b = i % NBUF
      gi[i] = pltpu.async_copy(y_hbm.at[src_v[b]], rows_v[b], s_g[b])

    issue_idx(0)
    if cpt > 1:
      issue_idx(1)
    di[0][0].wait()
    issue_gather(0)
    for i in range(cpt):
      if i + 1 < cpt:
        di[i + 1][0].wait()
        issue_gather(i + 1)
      gi[i].wait()
      di[i][1].wait()
      pltpu.sync_copy(rows_v[i % NBUF], acc_sh.at[dst_v[i % NBUF]], add=True)
      if i + 2 < cpt:
        issue_idx(i + 2)

    plsc.subcore_barrier()
    _copy_out(sid, cid, acc_sh, out_hbm, n, 40)

  return k(y, srcp, dstp)


def _decode_sc(z, ai, bi):
  n, d = z.shape
  elp = ai.shape[0]
  per_tile = elp // NW
  kc = 112
  cpt = per_tile // kc
  nj = d // 16
  nb = 2

  @functools.partial(
      pl.kernel,
      out_type=jax.ShapeDtypeStruct((elp, 16), jnp.float32),
      mesh=_sc_mesh(),
      scratch_types=(
          [pltpu.VMEM((kc,), jnp.int32) for _ in range(2 * nb)] +
          [pltpu.VMEM((kc, d), jnp.float32) for _ in range(2 * nb)] +
          [pltpu.VMEM((kc, 16), jnp.float32) for _ in range(nb)] +
          [pltpu.SemaphoreType.DMA for _ in range(5 * nb)]),
  )
  def k(z_hbm, a_hbm, b_hbm, out_hbm, *s):
    ai_v = s[:nb]
    bi_v = s[nb:2 * nb]
    za_v = s[2 * nb:3 * nb]
    zb_v = s[3 * nb:4 * nb]
    part_v = s[4 * nb:5 * nb]
    s_ai = s[5 * nb:6 * nb]
    s_bi = s[6 * nb:7 * nb]
    s_ga = s[7 * nb:8 * nb]
    s_gb = s[8 * nb:9 * nb]
    s_out = s[9 * nb:]
    cid = lax.axis_index("c")
    sid = lax.axis_index("s")
    wid = sid * NC + cid
    base = wid * per_tile
    di = {}
    gi = {}
    oi = {}

    def issue_idx(i):
      b = i % nb
      off = base + i * kc
      di[i] = (
          pltpu.async_copy(a_hbm.at[pl.ds(off, kc)], ai_v[b], s_ai[b]),
          pltpu.async_copy(b_hbm.at[pl.ds(off, kc)], bi_v[b], s_bi[b]))

    def issue_gathers(i):
      b = i % nb
      gi[i] = (
          pltpu.async_copy(z_hbm.at[ai_v[b]], za_v[b], s_ga[b]),
          pltpu.async_copy(z_hbm.at[bi_v[b]], zb_v[b], s_gb[b]))

    issue_idx(0)
    if cpt > 1:
      issue_idx(1)
    di[0][0].wait()
    di[0][1].wait()
    issue_gathers(0)
    for i in range(cpt):
      b = i % nb
      if i + 1 < cpt:
        di[i + 1][0].wait()
        di[i + 1][1].wait()
        issue_gathers(i + 1)
      gi[i][0].wait()
      gi[i][1].wait()
      if i - nb >= 0:
        oi[i - nb].wait()

      def ebody(ei, ecarry):
        acc = za_v[b][ei, pl.ds(0, 16)] * zb_v[b][ei, pl.ds(0, 16)]
        for j in range(1, nj):
          acc = acc + (za_v[b][ei, pl.ds(16 * j, 16)] *
                       zb_v[b][ei, pl.ds(16 * j, 16)])
        part_v[b][ei, :] = acc
        return ecarry

      lax.fori_loop(0, kc, ebody, 0)
      oi[i] = pltpu.async_copy(
          part_v[b], out_hbm.at[pl.ds(base + i * kc, kc)], s_out[b])
      if i + 2 < cpt:
        issue_idx(i + 2)
    for i in range(max(cpt - nb, 0), cpt):
      oi[i].wait()

  return k(z, ai, bi)


def _tc_mm1(deg2, x, w1):
  n, d = x.shape
  rb = 1000
  g = n // rb

  def body(dega, degb, x_ref, w_ref, y_ref):
    deg = dega[:, 0:1] + degb[:, 0:1] + 1.0
    dinv = lax.rsqrt(deg)
    y_ref[...] = dinv * jnp.dot(x_ref[...], w_ref[...],
                                preferred_element_type=jnp.float32)

  return pl.pallas_call(
      body,
      grid=(g,),
      in_specs=[
          pl.BlockSpec((rb, d), lambda i: (i, 0)),
          pl.BlockSpec((rb, d), lambda i: (i + g, 0)),
          pl.BlockSpec((rb, d), lambda i: (i, 0)),
          pl.BlockSpec((d, d), lambda i: (0, 0)),
      ],
      out_specs=pl.BlockSpec((rb, d), lambda i: (i, 0)),
      out_shape=jax.ShapeDtypeStruct((n, d), jnp.float32),
  )(deg2, deg2, x, w1)


def _tc_mm2(deg2, acc2, y1, b1, w2):
  n, d = y1.shape
  rb = 1000
  g = n // rb

  def body(dega, degb, acca, accb, y_ref, b_ref, w_ref, out_ref):
    deg = dega[:, 0:1] + degb[:, 0:1] + 1.0
    dinv = lax.rsqrt(deg)
    sums = (acca[...] + accb[...] + y_ref[...]) * dinv + b_ref[...]
    h = jnp.maximum(sums, 0.0)
    out_ref[...] = dinv * jnp.dot(h, w_ref[...],
                                  preferred_element_type=jnp.float32)

  return pl.pallas_call(
      body,
      grid=(g,),
      in_specs=[
          pl.BlockSpec((rb, d), lambda i: (i, 0)),
          pl.BlockSpec((rb, d), lambda i: (i + g, 0)),
          pl.BlockSpec((rb, d), lambda i: (i, 0)),
          pl.BlockSpec((rb, d), lambda i: (i + g, 0)),
          pl.BlockSpec((rb, d), lambda i: (i, 0)),
          pl.BlockSpec((1, d), lambda i: (0, 0)),
          pl.BlockSpec((d, d), lambda i: (0, 0)),
      ],
      out_specs=pl.BlockSpec((rb, d), lambda i: (i, 0)),
      out_shape=jax.ShapeDtypeStruct((n, d), jnp.float32),
  )(deg2, deg2, acc2, acc2, y1, b1, w2)


def _tc_fin(deg2, acc2, y2, b2):
  n, d = y2.shape
  rb = 1000
  g = n // rb

  def body(dega, degb, acca, accb, y_ref, b_ref, out_ref):
    deg = dega[:, 0:1] + degb[:, 0:1] + 1.0
    dinv = lax.rsqrt(deg)
    out_ref[...] = (acca[...] + accb[...] + y_ref[...]) * dinv + b_ref[...]

  return pl.pallas_call(
      body,
      grid=(g,),
      in_specs=[
          pl.BlockSpec((rb, d), lambda i: (i, 0)),
          pl.BlockSpec((rb, d), lambda i: (i + g, 0)),
          pl.BlockSpec((rb, d), lambda i: (i, 0)),
          pl.BlockSpec((rb, d), lambda i: (i + g, 0)),
          pl.BlockSpec((rb, d), lambda i: (i, 0)),
          pl.BlockSpec((1, d), lambda i: (0, 0)),
      ],
      out_specs=pl.BlockSpec((rb, d), lambda i: (i, 0)),
      out_shape=jax.ShapeDtypeStruct((n, d), jnp.float32),
  )(deg2, deg2, acc2, acc2, y2, b2)


def _tc_lanesum(part):
  elp = part.shape[0]
  g = 32
  rb = elp // g

  def body(p_ref, out_ref):
    out_ref[...] = jnp.sum(p_ref[...], axis=1, keepdims=True)

  return pl.pallas_call(
      body,
      grid=(g,),
      in_specs=[pl.BlockSpec((rb, 16), lambda i: (i, 0))],
      out_specs=pl.BlockSpec((rb, 1), lambda i: (i, 0)),
      out_shape=jax.ShapeDtypeStruct((elp, 1), jnp.float32),
  )(part)


def kernel(x, edge_index, edge_label_index, W1, b1, W2, b2):
  n, d = x.shape
  e = edge_index.shape[1]
  el = edge_label_index.shape[1]
  src = edge_index[0].astype(jnp.int32)
  dst = edge_index[1].astype(jnp.int32)

  cpt = -(-e // (NW * CHUNK))
  ep = cpt * NW * CHUNK
  epad = ep - e
  fill = jnp.arange(epad, dtype=jnp.int32)
  srcp = jnp.concatenate([src, n + (fill % 8)])
  dstp_agg = jnp.concatenate([dst, fill % n])
  dstp_deg = jnp.concatenate([dst, n + (fill % 8)])
  zpad = jnp.zeros((8, d), jnp.float32)

  kc = 112
  per_tile = -(-el // NW)
  per_tile = -(-per_tile // kc) * kc
  elp = per_tile * NW
  pad = elp - el
  ai = jnp.concatenate(
      [edge_label_index[0].astype(jnp.int32), jnp.zeros((pad,), jnp.int32)])
  bi = jnp.concatenate(
      [edge_label_index[1].astype(jnp.int32), jnp.zeros((pad,), jnp.int32)])

  ones2d = jnp.ones((CHUNK, d), jnp.float32)
  deg2 = _deg_sc(dstp_deg, ones2d, n)
  y1 = _tc_mm1(deg2, x, W1)
  acc1 = _agg_sc(jnp.concatenate([y1, zpad]), srcp, dstp_agg, n)
  y2 = _tc_mm2(deg2, acc1, y1, b1.reshape(1, d), W2)
  acc2 = _agg_sc(jnp.concatenate([y2, zpad]), srcp, dstp_agg, n)
  z = _tc_fin(deg2, acc2, y2, b2.reshape(1, d))
  part = _decode_sc(z, ai, bi)
  logits = _tc_lanesum(part)
  return logits[:el, 0]


# speedup vs baseline: 1.5988x; 1.0213x over previous
import functools

import jax
import jax.numpy as jnp
from jax import lax
from jax.experimental import pallas as pl
from jax.experimental.pallas import tpu as pltpu
from jax.experimental.pallas import tpu_sc as plsc

NC = 2
NS = 16
NW = NC * NS
CHUNK = 128
NBUF = 3


def _sc_mesh():
  return plsc.VectorSubcoreMesh(
      core_axis_name="c", subcore_axis_name="s",
      num_cores=NC, num_subcores=NS)


def _zero16():
  return jnp.zeros((16,), jnp.float32)


def _zero_table(sid, zero_v, acc_sh, n, zrows):
  zchunks = n // zrows
  ziters = -(-zchunks // NS)

  def zbody(i, carry):
    c = sid + NS * i

    @pl.when(c < zchunks)
    def _():
      pltpu.sync_copy(zero_v, acc_sh.at[pl.ds(c * zrows, zrows)])
    return carry

  lax.fori_loop(0, ziters, zbody, 0)


def _copy_out(sid, cid, acc_sh, out_hbm, n, crows):
  cchunks = n // crows
  citers = -(-cchunks // NS)

  def obody(i, carry):
    c = sid + NS * i

    @pl.when(c < cchunks)
    def _():
      pltpu.sync_copy(acc_sh.at[pl.ds(c * crows, crows)],
                      out_hbm.at[pl.ds(cid * n + c * crows, crows)])
    return carry

  lax.fori_loop(0, citers, obody, 0)


def _deg_sc(dstp, ones2d, n):
  d = ones2d.shape[1]
  ep = dstp.shape[0]
  e_half = ep // NC
  cpt = e_half // (NS * CHUNK)
  zrows = 8

  @functools.partial(
      pl.kernel,
      out_type=jax.ShapeDtypeStruct((NC * n, d), jnp.float32),
      mesh=_sc_mesh(),
      scratch_types=(
          [pltpu.VMEM((CHUNK,), jnp.int32) for _ in range(NBUF)] +
          [pltpu.VMEM((CHUNK, d), jnp.float32),
           pltpu.VMEM((zrows, d), jnp.float32),
           pltpu.VMEM_SHARED((n + 8, d), jnp.float32)] +
          [pltpu.SemaphoreType.DMA for _ in range(2 * NBUF)]),
  )
  def k(dst_hbm, ones_hbm, out_hbm, *s):
    dst_v = s[:NBUF]
    ones_v, zero_v, acc_sh = s[NBUF:NBUF + 3]
    s_di = s[NBUF + 3:NBUF + 3 + NBUF]
    s_sc = s[NBUF + 3 + NBUF:]
    cid = lax.axis_index("c")
    sid = lax.axis_index("s")
    z16 = _zero16()
    for r in range(zrows):
      for j in range(d // 16):
        zero_v[r, pl.ds(16 * j, 16)] = z16
    pltpu.sync_copy(ones_hbm, ones_v)
    _zero_table(sid, zero_v, acc_sh, n, zrows)
    plsc.subcore_barrier()

    base = cid * e_half + sid * cpt * CHUNK
    di = {}

    def issue_idx(i):
      b = i % NBUF
      di[i] = pltpu.async_copy(
          dst_hbm.at[pl.ds(base + i * CHUNK, CHUNK)], dst_v[b], s_di[b])

    si = {}
    issue_idx(0)
    if cpt > 1:
      issue_idx(1)
    for i in range(cpt):
      di[i].wait()
      b = i % NBUF
      si[i] = pltpu.async_copy(ones_v, acc_sh.at[dst_v[b]], s_sc[b],
                               add=True)
      if i + 2 < cpt:
        if i - 1 >= 0:
          si[i - 1].wait()
        issue_idx(i + 2)
    for i in range(max(cpt - 3, 0), cpt):
      si[i].wait()

    plsc.subcore_barrier()
    _copy_out(sid, cid, acc_sh, out_hbm, n, 40)

  return k(dstp, ones2d)


def _agg_sc(y, srcp, dstp, n):
  d = y.shape[1]
  ep = srcp.shape[0]
  e_half = ep // NC
  cpt = e_half // (NS * CHUNK)
  zrows = 8

  @functools.partial(
      pl.kernel,
      out_type=jax.ShapeDtypeStruct((NC * n, d), jnp.float32),
      mesh=_sc_mesh(),
      scratch_types=(
          [pltpu.VMEM((CHUNK,), jnp.int32) for _ in range(2 * NBUF)] +
          [pltpu.VMEM((CHUNK, d), jnp.float32) for _ in range(NBUF)] +
          [pltpu.VMEM((zrows, d), jnp.float32),
           pltpu.VMEM_SHARED((n, d), jnp.float32)] +
          [pltpu.SemaphoreType.DMA for _ in range(4 * NBUF)]),
  )
  def k(y_hbm, src_hbm, dst_hbm, out_hbm, *s):
    src_v = s[:NBUF]
    dst_v = s[NBUF:2 * NBUF]
    rows_v = s[2 * NBUF:3 * NBUF]
    zero_v, acc_sh = s[3 * NBUF:3 * NBUF + 2]
    s_si = s[3 * NBUF + 2:4 * NBUF + 2]
    s_di = s[4 * NBUF + 2:5 * NBUF + 2]
    s_g = s[5 * NBUF + 2:6 * NBUF + 2]
    s_sc = s[6 * NBUF + 2:]
    cid = lax.axis_index("c")
    sid = lax.axis_index("s")
    z16 = _zero16()
    for r in range(zrows):
      for j in range(d // 16):
        zero_v[r, pl.ds(16 * j, 16)] = z16
    _zero_table(sid, zero_v, acc_sh, n, zrows)
    plsc.subcore_barrier()

    base = cid * e_half + sid * cpt * CHUNK
    di = {}
    gi = {}

    def issue_idx(i):
      b = i % NBUF
      off = base + i * CHUNK
      di[i] = (
          pltpu.async_copy(src_hbm.at[pl.ds(off, CHUNK)], src_v[b], s_si[b]),
          pltpu.async_copy(dst_hbm.at[pl.ds(off, CHUNK)], dst_v[b], s_di[b]))

    def issue_gather(i):
      b = i % NBUF
      gi[i] = pltpu.async_copy(y_hbm.at[src_v[b]], rows_v[b], s_g[b])

    si = {}
    issue_idx(0)
    if cpt > 1:
      issue_idx(1)
    di[0][0].wait()
    issue_gather(0)
    for i in range(cpt):
      if i + 1 < cpt:
        di[i + 1][0].wait()
        issue_gather(i + 1)
      gi[i].wait()
      di[i][1].wait()
      b = i % NBUF
      si[i] = pltpu.async_copy(rows_v[b], acc_sh.at[dst_v[b]], s_sc[b],
                               add=True)
      if i + 2 < cpt:
        if i - 1 >= 0:
          si[i - 1].wait()
        issue_idx(i + 2)
    for i in range(max(cpt - 3, 0), cpt):
      si[i].wait()

    plsc.subcore_barrier()
    _copy_out(sid, cid, acc_sh, out_hbm, n, 40)

  return k(y, srcp, dstp)


def _decode_sc(z, ai, bi):
  n, d = z.shape
  elp = ai.shape[0]
  per_tile = elp // NW
  kc = 112
  cpt = per_tile // kc
  nj = d // 16
  nb = 2

  @functools.partial(
      pl.kernel,
      out_type=jax.ShapeDtypeStruct((elp, 16), jnp.float32),
      mesh=_sc_mesh(),
      scratch_types=(
          [pltpu.VMEM((kc,), jnp.int32) for _ in range(2 * nb)] +
          [pltpu.VMEM((kc, d), jnp.float32) for _ in range(2 * nb)] +
          [pltpu.VMEM((kc, 16), jnp.float32) for _ in range(nb)] +
          [pltpu.SemaphoreType.DMA for _ in range(5 * nb)]),
  )
  def k(z_hbm, a_hbm, b_hbm, out_hbm, *s):
    ai_v = s[:nb]
    bi_v = s[nb:2 * nb]
    za_v = s[2 * nb:3 * nb]
    zb_v = s[3 * nb:4 * nb]
    part_v = s[4 * nb:5 * nb]
    s_ai = s[5 * nb:6 * nb]
    s_bi = s[6 * nb:7 * nb]
    s_ga = s[7 * nb:8 * nb]
    s_gb = s[8 * nb:9 * nb]
    s_out = s[9 * nb:]
    cid = lax.axis_index("c")
    sid = lax.axis_index("s")
    wid = sid * NC + cid
    base = wid * per_tile
    di = {}
    gi = {}
    oi = {}

    def issue_idx(i):
      b = i % nb
      off = base + i * kc
      di[i] = (
          pltpu.async_copy(a_hbm.at[pl.ds(off, kc)], ai_v[b], s_ai[b]),
          pltpu.async_copy(b_hbm.at[pl.ds(off, kc)], bi_v[b], s_bi[b]))

    def issue_gathers(i):
      b = i % nb
      gi[i] = (
          pltpu.async_copy(z_hbm.at[ai_v[b]], za_v[b], s_ga[b]),
          pltpu.async_copy(z_hbm.at[bi_v[b]], zb_v[b], s_gb[b]))

    issue_idx(0)
    if cpt > 1:
      issue_idx(1)
    di[0][0].wait()
    di[0][1].wait()
    issue_gathers(0)
    for i in range(cpt):
      b = i % nb
      if i + 1 < cpt:
        di[i + 1][0].wait()
        di[i + 1][1].wait()
        issue_gathers(i + 1)
      gi[i][0].wait()
      gi[i][1].wait()
      if i - nb >= 0:
        oi[i - nb].wait()

      def ebody(ei, ecarry):
        acc = za_v[b][ei, pl.ds(0, 16)] * zb_v[b][ei, pl.ds(0, 16)]
        for j in range(1, nj):
          acc = acc + (za_v[b][ei, pl.ds(16 * j, 16)] *
                       zb_v[b][ei, pl.ds(16 * j, 16)])
        part_v[b][ei, :] = acc
        return ecarry

      lax.fori_loop(0, kc, ebody, 0, unroll=2)
      oi[i] = pltpu.async_copy(
          part_v[b], out_hbm.at[pl.ds(base + i * kc, kc)], s_out[b])
      if i + 2 < cpt:
        issue_idx(i + 2)
    for i in range(max(cpt - nb, 0), cpt):
      oi[i].wait()

  return k(z, ai, bi)


def _tc_mm1(deg2, x, w1):
  n, d = x.shape
  rb = 1000
  g = n // rb

  def body(dega, degb, x_ref, w_ref, y_ref):
    deg = dega[:, 0:1] + degb[:, 0:1] + 1.0
    dinv = lax.rsqrt(deg)
    y_ref[...] = dinv * jnp.dot(x_ref[...], w_ref[...],
                                preferred_element_type=jnp.float32)

  return pl.pallas_call(
      body,
      grid=(g,),
      in_specs=[
          pl.BlockSpec((rb, d), lambda i: (i, 0)),
          pl.BlockSpec((rb, d), lambda i: (i + g, 0)),
          pl.BlockSpec((rb, d), lambda i: (i, 0)),
          pl.BlockSpec((d, d), lambda i: (0, 0)),
      ],
      out_specs=pl.BlockSpec((rb, d), lambda i: (i, 0)),
      out_shape=jax.ShapeDtypeStruct((n, d), jnp.float32),
  )(deg2, deg2, x, w1)


def _tc_mm2(deg2, acc2, y1, b1, w2):
  n, d = y1.shape
  rb = 1000
  g = n // rb

  def body(dega, degb, acca, accb, y_ref, b_ref, w_ref, out_ref):
    deg = dega[:, 0:1] + degb[:, 0:1] + 1.0
    dinv = lax.rsqrt(deg)
    sums = (acca[...] + accb[...] + y_ref[...]) * dinv + b_ref[...]
    h = jnp.maximum(sums, 0.0)
    out_ref[...] = dinv * jnp.dot(h, w_ref[...],
                                  preferred_element_type=jnp.float32)

  return pl.pallas_call(
      body,
      grid=(g,),
      in_specs=[
          pl.BlockSpec((rb, d), lambda i: (i, 0)),
          pl.BlockSpec((rb, d), lambda i: (i + g, 0)),
          pl.BlockSpec((rb, d), lambda i: (i, 0)),
          pl.BlockSpec((rb, d), lambda i: (i + g, 0)),
          pl.BlockSpec((rb, d), lambda i: (i, 0)),
          pl.BlockSpec((1, d), lambda i: (0, 0)),
          pl.BlockSpec((d, d), lambda i: (0, 0)),
      ],
      out_specs=pl.BlockSpec((rb, d), lambda i: (i, 0)),
      out_shape=jax.ShapeDtypeStruct((n, d), jnp.float32),
  )(deg2, deg2, acc2, acc2, y1, b1, w2)


def _tc_fin(deg2, acc2, y2, b2):
  n, d = y2.shape
  rb = 1000
  g = n // rb

  def body(dega, degb, acca, accb, y_ref, b_ref, out_ref):
    deg = dega[:, 0:1] + degb[:, 0:1] + 1.0
    dinv = lax.rsqrt(deg)
    out_ref[...] = (acca[...] + accb[...] + y_ref[...]) * dinv + b_ref[...]

  return pl.pallas_call(
      body,
      grid=(g,),
      in_specs=[
          pl.BlockSpec((rb, d), lambda i: (i, 0)),
          pl.BlockSpec((rb, d), lambda i: (i + g, 0)),
          pl.BlockSpec((rb, d), lambda i: (i, 0)),
          pl.BlockSpec((rb, d), lambda i: (i + g, 0)),
          pl.BlockSpec((rb, d), lambda i: (i, 0)),
          pl.BlockSpec((1, d), lambda i: (0, 0)),
      ],
      out_specs=pl.BlockSpec((rb, d), lambda i: (i, 0)),
      out_shape=jax.ShapeDtypeStruct((n, d), jnp.float32),
  )(deg2, deg2, acc2, acc2, y2, b2)


def _tc_lanesum(part):
  elp = part.shape[0]
  g = 32
  rb = elp // g

  def body(p_ref, out_ref):
    out_ref[...] = jnp.sum(p_ref[...], axis=1, keepdims=True)

  return pl.pallas_call(
      body,
      grid=(g,),
      in_specs=[pl.BlockSpec((rb, 16), lambda i: (i, 0))],
      out_specs=pl.BlockSpec((rb, 1), lambda i: (i, 0)),
      out_shape=jax.ShapeDtypeStruct((elp, 1), jnp.float32),
  )(part)


def kernel(x, edge_index, edge_label_index, W1, b1, W2, b2):
  n, d = x.shape
  e = edge_index.shape[1]
  el = edge_label_index.shape[1]
  src = edge_index[0].astype(jnp.int32)
  dst = edge_index[1].astype(jnp.int32)

  cpt = -(-e // (NW * CHUNK))
  ep = cpt * NW * CHUNK
  epad = ep - e
  fill = jnp.arange(epad, dtype=jnp.int32)
  srcp = jnp.concatenate([src, n + (fill % 8)])
  dstp_agg = jnp.concatenate([dst, fill % n])
  dstp_deg = jnp.concatenate([dst, n + (fill % 8)])
  zpad = jnp.zeros((8, d), jnp.float32)

  kc = 112
  per_tile = -(-el // NW)
  per_tile = -(-per_tile // kc) * kc
  elp = per_tile * NW
  pad = elp - el
  ai = jnp.concatenate(
      [edge_label_index[0].astype(jnp.int32), jnp.zeros((pad,), jnp.int32)])
  bi = jnp.concatenate(
      [edge_label_index[1].astype(jnp.int32), jnp.zeros((pad,), jnp.int32)])

  ones2d = jnp.ones((CHUNK, d), jnp.float32)
  deg2 = _deg_sc(dstp_deg, ones2d, n)
  y1 = _tc_mm1(deg2, x, W1)
  acc1 = _agg_sc(jnp.concatenate([y1, zpad]), srcp, dstp_agg, n)
  y2 = _tc_mm2(deg2, acc1, y1, b1.reshape(1, d), W2)
  acc2 = _agg_sc(jnp.concatenate([y2, zpad]), srcp, dstp_agg, n)
  z = _tc_fin(deg2, acc2, y2, b2.reshape(1, d))
  part = _decode_sc(z, ai, bi)
  logits = _tc_lanesum(part)
  return logits[:el, 0]


# speedup vs baseline: 1.6257x; 1.0168x over previous
import functools

import jax
import jax.numpy as jnp
from jax import lax
from jax.experimental import pallas as pl
from jax.experimental.pallas import tpu as pltpu
from jax.experimental.pallas import tpu_sc as plsc

NC = 2
NS = 16
NW = NC * NS
CHUNK = 128
NBUF = 3


def _sc_mesh():
  return plsc.VectorSubcoreMesh(
      core_axis_name="c", subcore_axis_name="s",
      num_cores=NC, num_subcores=NS)


def _zero16():
  return jnp.zeros((16,), jnp.float32)


def _zero_table(sid, zero_v, acc_sh, n, zrows):
  zchunks = n // zrows
  ziters = -(-zchunks // NS)

  def zbody(i, carry):
    c = sid + NS * i

    @pl.when(c < zchunks)
    def _():
      pltpu.sync_copy(zero_v, acc_sh.at[pl.ds(c * zrows, zrows)])
    return carry

  lax.fori_loop(0, ziters, zbody, 0)


def _copy_out(sid, cid, acc_sh, out_hbm, n, crows):
  cchunks = n // crows
  citers = -(-cchunks // NS)

  def obody(i, carry):
    c = sid + NS * i

    @pl.when(c < cchunks)
    def _():
      pltpu.sync_copy(acc_sh.at[pl.ds(c * crows, crows)],
                      out_hbm.at[pl.ds(cid * n + c * crows, crows)])
    return carry

  lax.fori_loop(0, citers, obody, 0)


def _deg_sc(dstp, ones2d, n):
  d = ones2d.shape[1]
  ep = dstp.shape[0]
  e_half = ep // NC
  cpt = e_half // (NS * CHUNK)
  zrows = 8

  @functools.partial(
      pl.kernel,
      out_type=jax.ShapeDtypeStruct((NC * n, d), jnp.float32),
      mesh=_sc_mesh(),
      scratch_types=(
          [pltpu.VMEM((CHUNK,), jnp.int32) for _ in range(NBUF)] +
          [pltpu.VMEM((CHUNK, d), jnp.float32),
           pltpu.VMEM((zrows, d), jnp.float32),
           pltpu.VMEM_SHARED((n + 8, d), jnp.float32)] +
          [pltpu.SemaphoreType.DMA for _ in range(2 * NBUF)]),
  )
  def k(dst_hbm, ones_hbm, out_hbm, *s):
    dst_v = s[:NBUF]
    ones_v, zero_v, acc_sh = s[NBUF:NBUF + 3]
    s_di = s[NBUF + 3:NBUF + 3 + NBUF]
    s_sc = s[NBUF + 3 + NBUF:]
    cid = lax.axis_index("c")
    sid = lax.axis_index("s")
    z16 = _zero16()
    for r in range(zrows):
      for j in range(d // 16):
        zero_v[r, pl.ds(16 * j, 16)] = z16
    pltpu.sync_copy(ones_hbm, ones_v)
    _zero_table(sid, zero_v, acc_sh, n, zrows)
    plsc.subcore_barrier()

    base = cid * e_half + sid * cpt * CHUNK
    di = {}

    def issue_idx(i):
      b = i % NBUF
      di[i] = pltpu.async_copy(
          dst_hbm.at[pl.ds(base + i * CHUNK, CHUNK)], dst_v[b], s_di[b])

    si = {}
    issue_idx(0)
    if cpt > 1:
      issue_idx(1)
    for i in range(cpt):
      di[i].wait()
      b = i % NBUF
      si[i] = pltpu.async_copy(ones_v, acc_sh.at[dst_v[b]], s_sc[b],
                               add=True)
      if i + 2 < cpt:
        if i - 1 >= 0:
          si[i - 1].wait()
        issue_idx(i + 2)
    for i in range(max(cpt - 3, 0), cpt):
      si[i].wait()

    plsc.subcore_barrier()
    _copy_out(sid, cid, acc_sh, out_hbm, n, 40)

  return k(dstp, ones2d)


def _agg_sc(y, srcp, dstp, n):
  d = y.shape[1]
  ep = srcp.shape[0]
  e_half = ep // NC
  cpt = e_half // (NS * CHUNK)
  zrows = 8

  @functools.partial(
      pl.kernel,
      out_type=jax.ShapeDtypeStruct((NC * n, d), jnp.float32),
      mesh=_sc_mesh(),
      scratch_types=(
          [pltpu.VMEM((CHUNK,), jnp.int32) for _ in range(2 * NBUF)] +
          [pltpu.VMEM((CHUNK, d), jnp.float32) for _ in range(NBUF)] +
          [pltpu.VMEM((zrows, d), jnp.float32),
           pltpu.VMEM_SHARED((n, d), jnp.float32)] +
          [pltpu.SemaphoreType.DMA for _ in range(4 * NBUF)]),
  )
  def k(y_hbm, src_hbm, dst_hbm, out_hbm, *s):
    src_v = s[:NBUF]
    dst_v = s[NBUF:2 * NBUF]
    rows_v = s[2 * NBUF:3 * NBUF]
    zero_v, acc_sh = s[3 * NBUF:3 * NBUF + 2]
    s_si = s[3 * NBUF + 2:4 * NBUF + 2]
    s_di = s[4 * NBUF + 2:5 * NBUF + 2]
    s_g = s[5 * NBUF + 2:6 * NBUF + 2]
    s_sc = s[6 * NBUF + 2:]
    cid = lax.axis_index("c")
    sid = lax.axis_index("s")
    z16 = _zero16()
    for r in range(zrows):
      for j in range(d // 16):
        zero_v[r, pl.ds(16 * j, 16)] = z16
    _zero_table(sid, zero_v, acc_sh, n, zrows)
    plsc.subcore_barrier()

    base = cid * e_half + sid * cpt * CHUNK
    di = {}
    gi = {}

    def issue_idx(i):
      b = i % NBUF
      off = base + i * CHUNK
      di[i] = (
          pltpu.async_copy(src_hbm.at[pl.ds(off, CHUNK)], src_v[b], s_si[b]),
          pltpu.async_copy(dst_hbm.at[pl.ds(off, CHUNK)], dst_v[b], s_di[b]))

    def issue_gather(i):
      b = i % NBUF
      gi[i] = pltpu.async_copy(y_hbm.at[src_v[b]], rows_v[b], s_g[b])

    si = {}
    issue_idx(0)
    if cpt > 1:
      issue_idx(1)
    di[0][0].wait()
    issue_gather(0)
    for i in range(cpt):
      if i + 1 < cpt:
        di[i + 1][0].wait()
        issue_gather(i + 1)
      gi[i].wait()
      di[i][1].wait()
      b = i % NBUF
      si[i] = pltpu.async_copy(rows_v[b], acc_sh.at[dst_v[b]], s_sc[b],
                               add=True)
      if i + 2 < cpt:
        if i - 1 >= 0:
          si[i - 1].wait()
        issue_idx(i + 2)
    for i in range(max(cpt - 3, 0), cpt):
      si[i].wait()

    plsc.subcore_barrier()
    _copy_out(sid, cid, acc_sh, out_hbm, n, 40)

  return k(y, srcp, dstp)


def _decode_sc(z, ai, bi):
  n, d = z.shape
  elp = ai.shape[0]
  per_tile = elp // NW
  kc = 112
  cpt = per_tile // kc
  nj = d // 16
  nb = 2

  @functools.partial(
      pl.kernel,
      out_type=jax.ShapeDtypeStruct((elp, 16), jnp.float32),
      mesh=_sc_mesh(),
      scratch_types=(
          [pltpu.VMEM((kc,), jnp.int32) for _ in range(2 * nb)] +
          [pltpu.VMEM((kc, d), jnp.float32) for _ in range(2 * nb)] +
          [pltpu.VMEM((kc, 16), jnp.float32) for _ in range(nb)] +
          [pltpu.SemaphoreType.DMA for _ in range(5 * nb)]),
  )
  def k(z_hbm, a_hbm, b_hbm, out_hbm, *s):
    ai_v = s[:nb]
    bi_v = s[nb:2 * nb]
    za_v = s[2 * nb:3 * nb]
    zb_v = s[3 * nb:4 * nb]
    part_v = s[4 * nb:5 * nb]
    s_ai = s[5 * nb:6 * nb]
    s_bi = s[6 * nb:7 * nb]
    s_ga = s[7 * nb:8 * nb]
    s_gb = s[8 * nb:9 * nb]
    s_out = s[9 * nb:]
    cid = lax.axis_index("c")
    sid = lax.axis_index("s")
    wid = sid * NC + cid
    base = wid * per_tile
    di = {}
    gi = {}
    oi = {}

    def issue_idx(i):
      b = i % nb
      off = base + i * kc
      di[i] = (
          pltpu.async_copy(a_hbm.at[pl.ds(off, kc)], ai_v[b], s_ai[b]),
          pltpu.async_copy(b_hbm.at[pl.ds(off, kc)], bi_v[b], s_bi[b]))

    def issue_gathers(i):
      b = i % nb
      gi[i] = (
          pltpu.async_copy(z_hbm.at[ai_v[b]], za_v[b], s_ga[b]),
          pltpu.async_copy(z_hbm.at[bi_v[b]], zb_v[b], s_gb[b]))

    issue_idx(0)
    if cpt > 1:
      issue_idx(1)
    di[0][0].wait()
    di[0][1].wait()
    issue_gathers(0)
    for i in range(cpt):
      b = i % nb
      if i + 1 < cpt:
        di[i + 1][0].wait()
        di[i + 1][1].wait()
        issue_gathers(i + 1)
      gi[i][0].wait()
      gi[i][1].wait()
      if i - nb >= 0:
        oi[i - nb].wait()

      def ebody(ei, ecarry):
        acc = za_v[b][ei, pl.ds(0, 16)] * zb_v[b][ei, pl.ds(0, 16)]
        for j in range(1, nj):
          acc = acc + (za_v[b][ei, pl.ds(16 * j, 16)] *
                       zb_v[b][ei, pl.ds(16 * j, 16)])
        part_v[b][ei, :] = acc
        return ecarry

      lax.fori_loop(0, kc, ebody, 0)
      oi[i] = pltpu.async_copy(
          part_v[b], out_hbm.at[pl.ds(base + i * kc, kc)], s_out[b])
      if i + 2 < cpt:
        issue_idx(i + 2)
    for i in range(max(cpt - nb, 0), cpt):
      oi[i].wait()

  return k(z, ai, bi)


def _tc_mm1(deg2, x, w1):
  n, d = x.shape
  rb = 1000
  g = n // rb

  def body(dega, degb, x_ref, w_ref, y_ref):
    deg = dega[:, 0:1] + degb[:, 0:1] + 1.0
    dinv = lax.rsqrt(deg)
    y_ref[...] = dinv * jnp.dot(x_ref[...], w_ref[...],
                                preferred_element_type=jnp.float32)

  return pl.pallas_call(
      body,
      grid=(g,),
      in_specs=[
          pl.BlockSpec((rb, d), lambda i: (i, 0)),
          pl.BlockSpec((rb, d), lambda i: (i + g, 0)),
          pl.BlockSpec((rb, d), lambda i: (i, 0)),
          pl.BlockSpec((d, d), lambda i: (0, 0)),
      ],
      out_specs=pl.BlockSpec((rb, d), lambda i: (i, 0)),
      out_shape=jax.ShapeDtypeStruct((n, d), jnp.float32),
  )(deg2, deg2, x, w1)


def _tc_mm2(deg2, acc2, y1, b1, w2):
  n, d = y1.shape
  rb = 1000
  g = n // rb

  def body(dega, degb, acca, accb, y_ref, b_ref, w_ref, out_ref):
    deg = dega[:, 0:1] + degb[:, 0:1] + 1.0
    dinv = lax.rsqrt(deg)
    sums = (acca[...] + accb[...] + y_ref[...]) * dinv + b_ref[...]
    h = jnp.maximum(sums, 0.0)
    out_ref[...] = dinv * jnp.dot(h, w_ref[...],
                                  preferred_element_type=jnp.float32)

  return pl.pallas_call(
      body,
      grid=(g,),
      in_specs=[
          pl.BlockSpec((rb, d), lambda i: (i, 0)),
          pl.BlockSpec((rb, d), lambda i: (i + g, 0)),
          pl.BlockSpec((rb, d), lambda i: (i, 0)),
          pl.BlockSpec((rb, d), lambda i: (i + g, 0)),
          pl.BlockSpec((rb, d), lambda i: (i, 0)),
          pl.BlockSpec((1, d), lambda i: (0, 0)),
          pl.BlockSpec((d, d), lambda i: (0, 0)),
      ],
      out_specs=pl.BlockSpec((rb, d), lambda i: (i, 0)),
      out_shape=jax.ShapeDtypeStruct((n, d), jnp.float32),
  )(deg2, deg2, acc2, acc2, y1, b1, w2)


def _tc_fin(deg2, acc2, y2, b2):
  n, d = y2.shape
  rb = 1000
  g = n // rb

  def body(dega, degb, acca, accb, y_ref, b_ref, out_ref):
    deg = dega[:, 0:1] + degb[:, 0:1] + 1.0
    dinv = lax.rsqrt(deg)
    out_ref[...] = (acca[...] + accb[...] + y_ref[...]) * dinv + b_ref[...]

  return pl.pallas_call(
      body,
      grid=(g,),
      in_specs=[
          pl.BlockSpec((rb, d), lambda i: (i, 0)),
          pl.BlockSpec((rb, d), lambda i: (i + g, 0)),
          pl.BlockSpec((rb, d), lambda i: (i, 0)),
          pl.BlockSpec((rb, d), lambda i: (i + g, 0)),
          pl.BlockSpec((rb, d), lambda i: (i, 0)),
          pl.BlockSpec((1, d), lambda i: (0, 0)),
      ],
      out_specs=pl.BlockSpec((rb, d), lambda i: (i, 0)),
      out_shape=jax.ShapeDtypeStruct((n, d), jnp.float32),
  )(deg2, deg2, acc2, acc2, y2, b2)


def _tc_lanesum(part):
  elp = part.shape[0]
  g = 32
  rb = elp // g

  def body(p_ref, out_ref):
    out_ref[...] = jnp.sum(p_ref[...], axis=1, keepdims=True)

  return pl.pallas_call(
      body,
      grid=(g,),
      in_specs=[pl.BlockSpec((rb, 16), lambda i: (i, 0))],
      out_specs=pl.BlockSpec((rb, 1), lambda i: (i, 0)),
      out_shape=jax.ShapeDtypeStruct((elp, 1), jnp.float32),
  )(part)


def kernel(x, edge_index, edge_label_index, W1, b1, W2, b2):
  n, d = x.shape
  e = edge_index.shape[1]
  el = edge_label_index.shape[1]
  src = edge_index[0].astype(jnp.int32)
  dst = edge_index[1].astype(jnp.int32)

  cpt = -(-e // (NW * CHUNK))
  ep = cpt * NW * CHUNK
  epad = ep - e
  fill = jnp.arange(epad, dtype=jnp.int32)
  srcp = jnp.concatenate([src, n + (fill % 8)])
  dstp_agg = jnp.concatenate([dst, fill % n])
  dstp_deg = jnp.concatenate([dst, n + (fill % 8)])
  zpad = jnp.zeros((8, d), jnp.float32)

  kc = 112
  per_tile = -(-el // NW)
  per_tile = -(-per_tile // kc) * kc
  elp = per_tile * NW
  pad = elp - el
  ai = jnp.concatenate(
      [edge_label_index[0].astype(jnp.int32), jnp.zeros((pad,), jnp.int32)])
  bi = jnp.concatenate(
      [edge_label_index[1].astype(jnp.int32), jnp.zeros((pad,), jnp.int32)])

  ones2d = jnp.ones((CHUNK, d), jnp.float32)
  deg2 = _deg_sc(dstp_deg, ones2d, n)
  y1 = _tc_mm1(deg2, x, W1)
  acc1 = _agg_sc(jnp.concatenate([y1, zpad]), srcp, dstp_agg, n)
  y2 = _tc_mm2(deg2, acc1, y1, b1.reshape(1, d), W2)
  acc2 = _agg_sc(jnp.concatenate([y2, zpad]), srcp, dstp_agg, n)
  z = _tc_fin(deg2, acc2, y2, b2.reshape(1, d))
  part = _decode_sc(z, ai, bi)
  logits = _tc_lanesum(part)
  return logits[:el, 0]
